# baseline (speedup 1.0000x reference)
import functools

import jax
import jax.numpy as jnp
from jax import lax
from jax.experimental import pallas as pl
from jax.experimental.pallas import tpu as pltpu
from jax.experimental.pallas import tpu_sc as plsc

VOCAB = 1000000
EMBED = 64
FEAT = 128
N_FEAT_NODES = 8192
B = 16384

_info = plsc.get_sparse_core_info()
_NC, _NS = _info.num_cores, _info.num_subcores
_NW = _NC * _NS
_BPW = B // _NW
_C = 64
_NCH = _BPW // _C
_PPW = N_FEAT_NODES // _NW

_mesh = plsc.VectorSubcoreMesh(core_axis_name="c", subcore_axis_name="s")


@functools.partial(
    pl.kernel,
    mesh=_mesh,
    out_type=jax.ShapeDtypeStruct((N_FEAT_NODES + B, EMBED), jnp.float32),
    scratch_types=[
        pltpu.VMEM((_BPW,), jnp.int32),
        pltpu.VMEM((_BPW, EMBED), jnp.float32),
        pltpu.VMEM((_PPW, EMBED), jnp.float32),
        pltpu.SemaphoreType.DMA,
    ],
)
def _sc_assemble(table_hbm, idx_hbm, proj_hbm, out_hbm, idx_v, rows_v,
                 paper_v, sem):
    wid = lax.axis_index("s") * _NC + lax.axis_index("c")
    base = wid * _BPW
    pltpu.sync_copy(idx_hbm.at[pl.ds(base, _BPW)], idx_v)

    pcp = pltpu.async_copy(proj_hbm.at[pl.ds(wid * _PPW, _PPW)], paper_v,
                           sem)

    def fire_chunk(k):
        copies = []
        for g in range(_C // 16):
            v16 = idx_v[pl.ds(k * _C + g * 16, 16)]
            for l in range(16):
                nid = v16[l]
                t = lax.shift_right_logical(nid, 3)
                s = lax.bitwise_and(nid, 7)
                copies.append(
                    pltpu.async_copy(table_hbm.at[t, s],
                                     rows_v.at[k * _C + g * 16 + l], sem))
        return copies

    prev = fire_chunk(0)
    for k in range(1, _NCH):
        cur = fire_chunk(k)
        for cp in prev:
            cp.wait()
        prev = cur
    for cp in prev:
        cp.wait()
    pcp.wait()
    pltpu.sync_copy(paper_v, out_hbm.at[pl.ds(wid * _PPW, _PPW)])
    pltpu.sync_copy(rows_v, out_hbm.at[pl.ds(N_FEAT_NODES + base, _BPW)])


def _proj_body(h_ref, w_ref, o_ref):
    o_ref[...] = jnp.dot(h_ref[...], w_ref[...],
                         preferred_element_type=jnp.float32)


_proj = pl.pallas_call(
    _proj_body,
    out_shape=jax.ShapeDtypeStruct((N_FEAT_NODES, EMBED), jnp.float32),
)


def kernel(h_f_paper, nid_author, W_paper, E_author):
    emb_paper = _proj(h_f_paper, W_paper)
    tbl3 = E_author.reshape(VOCAB // 8, 8, EMBED)
    return _sc_assemble(tbl3, nid_author, emb_paper)


# speedup vs baseline: 1.0329x; 1.0329x over previous
import functools

import jax
import jax.numpy as jnp
from jax import lax
from jax.experimental import pallas as pl
from jax.experimental.pallas import tpu as pltpu
from jax.experimental.pallas import tpu_sc as plsc

VOCAB = 1000000
EMBED = 64
FEAT = 128
N_FEAT_NODES = 8192
B = 16384

_info = plsc.get_sparse_core_info()
_NC, _NS = _info.num_cores, _info.num_subcores
_NW = _NC * _NS
_BPW = B // _NW
_C = 64
_NCH = _BPW // _C
_PPW = N_FEAT_NODES // _NW

_mesh = plsc.VectorSubcoreMesh(core_axis_name="c", subcore_axis_name="s")


@functools.partial(
    pl.kernel,
    mesh=_mesh,
    out_type=jax.ShapeDtypeStruct((N_FEAT_NODES + B, EMBED), jnp.float32),
    scratch_types=[
        pltpu.VMEM((_BPW,), jnp.int32),
        pltpu.VMEM((_BPW, EMBED), jnp.float32),
        pltpu.VMEM((_PPW, EMBED), jnp.float32),
        pltpu.SemaphoreType.DMA,
    ],
)
def _sc_assemble(table_hbm, idx_hbm, proj_hbm, out_hbm, idx_v, rows_v,
                 paper_v, sem):
    wid = lax.axis_index("s") * _NC + lax.axis_index("c")
    base = wid * _BPW
    pltpu.sync_copy(idx_hbm.at[pl.ds(base, _BPW)], idx_v)

    pcp = pltpu.async_copy(proj_hbm.at[pl.ds(wid * _PPW, _PPW)], paper_v,
                           sem)

    def chunk_body(k, _):
        copies = []
        for g in range(_C // 16):
            v16 = idx_v[pl.ds(k * _C + g * 16, 16)]
            for l in range(16):
                nid = v16[l]
                t = lax.shift_right_logical(nid, 3)
                s = lax.bitwise_and(nid, 7)
                copies.append(
                    pltpu.async_copy(table_hbm.at[t, s],
                                     rows_v.at[k * _C + g * 16 + l], sem))
        for cp in copies:
            cp.wait()
        return 0

    lax.fori_loop(0, _NCH, chunk_body, 0)
    pcp.wait()
    pltpu.sync_copy(paper_v, out_hbm.at[pl.ds(wid * _PPW, _PPW)])
    pltpu.sync_copy(rows_v, out_hbm.at[pl.ds(N_FEAT_NODES + base, _BPW)])


def _proj_body(h_ref, w_ref, o_ref):
    o_ref[...] = jnp.dot(h_ref[...], w_ref[...],
                         preferred_element_type=jnp.float32)


_proj = pl.pallas_call(
    _proj_body,
    out_shape=jax.ShapeDtypeStruct((N_FEAT_NODES, EMBED), jnp.float32),
)


def kernel(h_f_paper, nid_author, W_paper, E_author):
    emb_paper = _proj(h_f_paper, W_paper)
    tbl3 = E_author.reshape(VOCAB // 8, 8, EMBED)
    return _sc_assemble(tbl3, nid_author, emb_paper)
